# baseline (speedup 1.0000x reference)
import functools

import jax
import jax.numpy as jnp
from jax.experimental import pallas as pl

_ANCHORS_W = (116.0, 156.0, 373.0)
_ANCHORS_H = (90.0, 198.0, 326.0)
_IMG_SIZE = 608


def _decode_kernel(in_ref, out_ref, *, G, stride, BB):
    S = G * G
    lane = jax.lax.broadcasted_iota(jnp.int32, (1, S), 1)
    grid_x = (lane % G).astype(jnp.float32)
    grid_y = (lane // G).astype(jnp.float32)
    for b in range(BB):
        for a in range(3):
            t = in_ref[b, a]
            sig = jax.nn.sigmoid(t)
            out_ref[b, a] = sig.T
            bx = (sig[0:1] + grid_x) * stride
            by = (sig[1:2] + grid_y) * stride
            bw = jnp.exp(t[2:3]) * _ANCHORS_W[a]
            bh = jnp.exp(t[3:4]) * _ANCHORS_H[a]
            boxes = jnp.concatenate([bx, by, bw, bh], axis=0)
            out_ref[b, a, :, 0:4] = boxes.T


def kernel(prediction):
    B, C, G, _ = prediction.shape
    nA = 3
    attrs = C // nA
    S = G * G
    stride = _IMG_SIZE // G
    BB = 1
    pred2 = prediction.reshape(B, nA, attrs, S)
    out = pl.pallas_call(
        functools.partial(_decode_kernel, G=G, stride=float(stride), BB=BB),
        grid=(B // BB,),
        in_specs=[pl.BlockSpec((BB, nA, attrs, S), lambda b: (b, 0, 0, 0))],
        out_specs=pl.BlockSpec((BB, nA, S, attrs), lambda b: (b, 0, 0, 0)),
        out_shape=jax.ShapeDtypeStruct((B, nA, S, attrs), jnp.float32),
    )(pred2)
    return out.reshape(B, nA * S, attrs)


# speedup vs baseline: 2.0746x; 2.0746x over previous
import functools

import jax
import jax.numpy as jnp
from jax.experimental import pallas as pl

_ANCHORS_W = (116.0, 156.0, 373.0)
_ANCHORS_H = (90.0, 198.0, 326.0)
_IMG_SIZE = 608


def _decode_kernel(in_ref, out_ref, *, G, stride):
    S = G * G
    lane = jax.lax.broadcasted_iota(jnp.int32, (1, S), 1)
    grid_x = (lane % G).astype(jnp.float32)
    grid_y = (lane // G).astype(jnp.float32)
    for a in range(3):
        t = in_ref[0, 85 * a:85 * (a + 1), :]
        sig = jax.nn.sigmoid(t)
        out_ref[0, a] = sig.T
        bx = (sig[0:1] + grid_x) * stride
        by = (sig[1:2] + grid_y) * stride
        bw = jnp.exp(t[2:3]) * _ANCHORS_W[a]
        bh = jnp.exp(t[3:4]) * _ANCHORS_H[a]
        boxes = jnp.concatenate([bx, by, bw, bh], axis=0)
        out_ref[0, a, :, 0:4] = boxes.T


def kernel(prediction):
    B, C, G, _ = prediction.shape
    nA = 3
    attrs = C // nA
    S = G * G
    stride = _IMG_SIZE // G
    pred2 = prediction.reshape(B, C, S)
    out = pl.pallas_call(
        functools.partial(_decode_kernel, G=G, stride=float(stride)),
        grid=(B,),
        in_specs=[pl.BlockSpec((1, C, S), lambda b: (b, 0, 0))],
        out_specs=pl.BlockSpec((1, nA, S, attrs), lambda b: (b, 0, 0, 0)),
        out_shape=jax.ShapeDtypeStruct((B, nA, S, attrs), jnp.float32),
    )(pred2)
    return out.reshape(B, nA * S, attrs)


# speedup vs baseline: 2.1118x; 1.0179x over previous
import functools

import jax
import jax.numpy as jnp
from jax.experimental import pallas as pl

_ANCHORS_W = (116.0, 156.0, 373.0)
_ANCHORS_H = (90.0, 198.0, 326.0)
_IMG_SIZE = 608


def _decode_kernel(in_ref, out_ref, *, G, stride):
    S = G * G
    lane = jax.lax.broadcasted_iota(jnp.int32, (1, S), 1)
    grid_x = (lane % G).astype(jnp.float32)
    grid_y = (lane // G).astype(jnp.float32)
    for b in range(2):
      for a in range(3):
        t = in_ref[b, 85 * a:85 * (a + 1), :]
        sig = jax.nn.sigmoid(t)
        out_ref[b, a] = sig.T
        bx = (sig[0:1] + grid_x) * stride
        by = (sig[1:2] + grid_y) * stride
        bw = jnp.exp(t[2:3]) * _ANCHORS_W[a]
        bh = jnp.exp(t[3:4]) * _ANCHORS_H[a]
        boxes = jnp.concatenate([bx, by, bw, bh], axis=0)
        out_ref[b, a, :, 0:4] = boxes.T


def kernel(prediction):
    B, C, G, _ = prediction.shape
    nA = 3
    attrs = C // nA
    S = G * G
    stride = _IMG_SIZE // G
    pred2 = prediction.reshape(B, C, S)
    out = pl.pallas_call(
        functools.partial(_decode_kernel, G=G, stride=float(stride)),
        grid=(B // 2,),
        in_specs=[pl.BlockSpec((2, C, S), lambda b: (b, 0, 0))],
        out_specs=pl.BlockSpec((2, nA, S, attrs), lambda b: (b, 0, 0, 0)),
        out_shape=jax.ShapeDtypeStruct((B, nA, S, attrs), jnp.float32),
    )(pred2)
    return out.reshape(B, nA * S, attrs)
